# baseline (speedup 1.0000x reference)
import jax
import jax.numpy as jnp
from jax import lax
from jax.experimental import pallas as pl
from jax.experimental.pallas import tpu as pltpu

_B, _F, _N, _D = 8, 2, 1024, 16
_K = 20
_MM = jnp.bfloat16
_DN = (((1,), (1,)), ((), ()))

_NOISE = None


def _noise_const():
    global _NOISE
    if _NOISE is not None:
        return _NOISE
    import numpy as np

    def rotl(x, r):
        return ((x << np.uint32(r)) | (x >> np.uint32(32 - r))).astype(np.uint32)

    n = _B * _N * _N
    i = np.arange(n, dtype=np.uint64)
    x0 = (i >> np.uint64(32)).astype(np.uint32)
    x1 = (i & np.uint64(0xFFFFFFFF)).astype(np.uint32)
    k0, k1 = np.uint32(0), np.uint32(42)
    ks = [k0, k1, np.uint32(k0 ^ k1 ^ np.uint32(0x1BD11BDA))]
    rotations = [[13, 15, 26, 6], [17, 29, 16, 24]]
    x0 = (x0 + ks[0]).astype(np.uint32)
    x1 = (x1 + ks[1]).astype(np.uint32)
    for r in range(5):
        for rot in rotations[r % 2]:
            x0 = (x0 + x1).astype(np.uint32)
            x1 = rotl(x1, rot) ^ x0
        x0 = (x0 + ks[(r + 1) % 3]).astype(np.uint32)
        x1 = (x1 + ks[(r + 2) % 3] + np.uint32(r + 1)).astype(np.uint32)
    bits = x0 ^ x1
    f = ((bits >> np.uint32(9)) | np.uint32(0x3F800000)).view(np.float32)
    f = np.maximum(np.float32(0.0), f - np.float32(1.0))
    _NOISE = (f * np.float32(0.01)).reshape(_B, _N, _N)
    return _NOISE


def _body(x1_ref, x2_ref, noise_ref, out_ref,
          pert_ref, adj_ref, nv1_ref, nv2_ref):
    s = pl.program_id(0)
    cur = s % 2
    prv = (s + 1) % 2

    @pl.when(s < _B)
    def _():
        def nodevec(xref):
            c = jnp.concatenate([xref[0, 0], xref[0, 1]], axis=1).astype(_MM)
            sim = lax.dot_general(c, c, _DN, preferred_element_type=jnp.float32)
            return jnp.tanh(sim * 0.5).astype(_MM)

        nv1_ref[...] = nodevec(x1_ref)
        nv2_ref[...] = nodevec(x2_ref)

    def mm_pass(t):
        parts = []
        for c in range(16):
            w = pert_ref[prv, c * 64:(c + 1) * 64, :].reshape(8, 8, _N)
            w = jnp.where(w >= t, -jnp.inf, w)
            parts.append(jnp.max(w, axis=0))
        while len(parts) > 1:
            parts = [jnp.maximum(parts[i], parts[i + 1])
                     for i in range(0, len(parts), 2)]
        return jnp.max(parts[0], axis=0, keepdims=True).reshape(1, 1, _N)

    def step_build(p, t):
        t_new = mm_pass(t)
        c0 = p * 256
        nv1c = nv1_ref[pl.ds(c0, 256), :]
        nv2c = nv2_ref[pl.ds(c0, 256), :]
        adj = (lax.dot_general(nv1_ref[...], nv2c, _DN,
                               preferred_element_type=jnp.float32)
               - lax.dot_general(nv2_ref[...], nv1c, _DN,
                                 preferred_element_type=jnp.float32))
        adj_ref[cur, :, pl.ds(c0, 256)] = adj
        pert_ref[cur, :, pl.ds(c0, 256)] = adj + noise_ref[0, :, pl.ds(c0, 256)]
        return t_new

    t4 = lax.fori_loop(0, 4, step_build,
                       jnp.full((1, 1, _N), jnp.inf, jnp.float32))
    t20 = lax.fori_loop(4, _K, lambda p, t: mm_pass(t), t4)

    @pl.when(s > 0)
    def _():
        w = pert_ref[prv]
        out_ref[0] = jnp.where(w >= t20.reshape(1, _N), adj_ref[prv], 0.0)


def _run(x1, x2, noise):
    return pl.pallas_call(
        _body,
        grid=(_B + 1,),
        in_specs=[
            pl.BlockSpec((1, _F, _N, _D),
                         lambda s: (jnp.minimum(s, _B - 1), 0, 0, 0)),
            pl.BlockSpec((1, _F, _N, _D),
                         lambda s: (jnp.minimum(s, _B - 1), 0, 0, 0)),
            pl.BlockSpec((1, _N, _N),
                         lambda s: (jnp.minimum(s, _B - 1), 0, 0)),
        ],
        out_specs=pl.BlockSpec((1, _N, _N),
                               lambda s: (jnp.maximum(s - 1, 0), 0, 0)),
        out_shape=jax.ShapeDtypeStruct((_B, _N, _N), jnp.float32),
        scratch_shapes=[
            pltpu.VMEM((2, _N, _N), jnp.float32),
            pltpu.VMEM((2, _N, _N), jnp.float32),
            pltpu.VMEM((_N, _N), _MM),
            pltpu.VMEM((_N, _N), _MM),
        ],
    )(x1, x2, noise)


def kernel(idx, time_in_day_feat, day_in_week_feat, emb1_table, emb2_table):
    return _run(time_in_day_feat, day_in_week_feat, _noise_const())


# speedup vs baseline: 1.0508x; 1.0508x over previous
import jax
import jax.numpy as jnp
from jax import lax
from jax.experimental import pallas as pl
from jax.experimental.pallas import tpu as pltpu

_B, _F, _N, _D = 8, 2, 1024, 16
_K = 20
_MM = jnp.bfloat16
_DN = (((1,), (1,)), ((), ()))

_NOISE = None


def _noise_const():
    global _NOISE
    if _NOISE is not None:
        return _NOISE
    import numpy as np

    def rotl(x, r):
        return ((x << np.uint32(r)) | (x >> np.uint32(32 - r))).astype(np.uint32)

    n = _B * _N * _N
    i = np.arange(n, dtype=np.uint64)
    x0 = (i >> np.uint64(32)).astype(np.uint32)
    x1 = (i & np.uint64(0xFFFFFFFF)).astype(np.uint32)
    k0, k1 = np.uint32(0), np.uint32(42)
    ks = [k0, k1, np.uint32(k0 ^ k1 ^ np.uint32(0x1BD11BDA))]
    rotations = [[13, 15, 26, 6], [17, 29, 16, 24]]
    x0 = (x0 + ks[0]).astype(np.uint32)
    x1 = (x1 + ks[1]).astype(np.uint32)
    for r in range(5):
        for rot in rotations[r % 2]:
            x0 = (x0 + x1).astype(np.uint32)
            x1 = rotl(x1, rot) ^ x0
        x0 = (x0 + ks[(r + 1) % 3]).astype(np.uint32)
        x1 = (x1 + ks[(r + 2) % 3] + np.uint32(r + 1)).astype(np.uint32)
    bits = x0 ^ x1
    f = ((bits >> np.uint32(9)) | np.uint32(0x3F800000)).view(np.float32)
    f = np.maximum(np.float32(0.0), f - np.float32(1.0))
    _NOISE = (f * np.float32(0.01)).reshape(_B, _N, _N)
    return _NOISE


def _body(x1_ref, x2_ref, noise_ref, out_ref,
          pert_ref, adj_ref, nv1_ref, nv2_ref):
    s = pl.program_id(0)
    cur = s % 2
    prv = (s + 1) % 2

    @pl.when(s < _B)
    def _():
        def nodevec(xref):
            c = jnp.concatenate([xref[0, 0], xref[0, 1]], axis=1).astype(_MM)
            sim = lax.dot_general(c, c, _DN, preferred_element_type=jnp.float32)
            return jnp.tanh(sim * 0.5).astype(_MM)

        nv1_ref[...] = nodevec(x1_ref)
        nv2_ref[...] = nodevec(x2_ref)

    def mm_pass(t):
        parts = []
        for c in range(16):
            w = pert_ref[prv, c * 64:(c + 1) * 64, :].reshape(8, 8, _N)
            w = jnp.where(w >= t, -jnp.inf, w)
            parts.append(jnp.max(w, axis=0))
        while len(parts) > 1:
            parts = [jnp.maximum(parts[i], parts[i + 1])
                     for i in range(0, len(parts), 2)]
        return jnp.max(parts[0], axis=0, keepdims=True).reshape(1, 1, _N)

    def step_build(p, t):
        t_new = mm_pass(t)
        c0 = p * 256
        nv1c = nv1_ref[pl.ds(c0, 256), :]
        nv2c = nv2_ref[pl.ds(c0, 256), :]
        adj = (lax.dot_general(nv1c, nv2_ref[...], _DN,
                               preferred_element_type=jnp.float32)
               - lax.dot_general(nv2c, nv1_ref[...], _DN,
                                 preferred_element_type=jnp.float32))
        adj_ref[cur, pl.ds(c0, 256), :] = adj
        pert_ref[cur, pl.ds(c0, 256), :] = adj + noise_ref[0, pl.ds(c0, 256), :]
        return t_new

    t4 = lax.fori_loop(0, 4, step_build,
                       jnp.full((1, 1, _N), jnp.inf, jnp.float32))
    t20 = lax.fori_loop(4, _K, lambda p, t: mm_pass(t), t4)

    @pl.when(s > 0)
    def _():
        w = pert_ref[prv]
        out_ref[0] = jnp.where(w >= t20.reshape(1, _N), adj_ref[prv], 0.0)


def _run(x1, x2, noise):
    return pl.pallas_call(
        _body,
        grid=(_B + 1,),
        in_specs=[
            pl.BlockSpec((1, _F, _N, _D),
                         lambda s: (jnp.minimum(s, _B - 1), 0, 0, 0)),
            pl.BlockSpec((1, _F, _N, _D),
                         lambda s: (jnp.minimum(s, _B - 1), 0, 0, 0)),
            pl.BlockSpec((1, _N, _N),
                         lambda s: (jnp.minimum(s, _B - 1), 0, 0)),
        ],
        out_specs=pl.BlockSpec((1, _N, _N),
                               lambda s: (jnp.maximum(s - 1, 0), 0, 0)),
        out_shape=jax.ShapeDtypeStruct((_B, _N, _N), jnp.float32),
        scratch_shapes=[
            pltpu.VMEM((2, _N, _N), jnp.float32),
            pltpu.VMEM((2, _N, _N), jnp.float32),
            pltpu.VMEM((_N, _N), _MM),
            pltpu.VMEM((_N, _N), _MM),
        ],
    )(x1, x2, noise)


def kernel(idx, time_in_day_feat, day_in_week_feat, emb1_table, emb2_table):
    return _run(time_in_day_feat, day_in_week_feat, _noise_const())


# speedup vs baseline: 1.3066x; 1.2435x over previous
import jax
import jax.numpy as jnp
from jax import lax
from jax.experimental import pallas as pl
from jax.experimental.pallas import tpu as pltpu

_B, _F, _N, _D = 8, 2, 1024, 16
_K = 20
_MM = jnp.bfloat16
_DN = (((1,), (1,)), ((), ()))

_NOISE = None


def _noise_const():
    global _NOISE
    if _NOISE is not None:
        return _NOISE
    import numpy as np

    def rotl(x, r):
        return ((x << np.uint32(r)) | (x >> np.uint32(32 - r))).astype(np.uint32)

    n = _B * _N * _N
    i = np.arange(n, dtype=np.uint64)
    x0 = (i >> np.uint64(32)).astype(np.uint32)
    x1 = (i & np.uint64(0xFFFFFFFF)).astype(np.uint32)
    k0, k1 = np.uint32(0), np.uint32(42)
    ks = [k0, k1, np.uint32(k0 ^ k1 ^ np.uint32(0x1BD11BDA))]
    rotations = [[13, 15, 26, 6], [17, 29, 16, 24]]
    x0 = (x0 + ks[0]).astype(np.uint32)
    x1 = (x1 + ks[1]).astype(np.uint32)
    for r in range(5):
        for rot in rotations[r % 2]:
            x0 = (x0 + x1).astype(np.uint32)
            x1 = rotl(x1, rot) ^ x0
        x0 = (x0 + ks[(r + 1) % 3]).astype(np.uint32)
        x1 = (x1 + ks[(r + 2) % 3] + np.uint32(r + 1)).astype(np.uint32)
    bits = x0 ^ x1
    f = ((bits >> np.uint32(9)) | np.uint32(0x3F800000)).view(np.float32)
    f = np.maximum(np.float32(0.0), f - np.float32(1.0))
    _NOISE = (f * np.float32(0.01)).reshape(_B, _N, _N)
    return _NOISE


def _body(x1_ref, x2_ref, noise_ref, out_ref, work_ref):
    def nodevec(xref):
        c = jnp.concatenate([xref[0, 0], xref[0, 1]], axis=1).astype(_MM)
        s = lax.dot_general(c, c, _DN, preferred_element_type=jnp.float32)
        return jnp.tanh(s * 0.5)

    nv1 = nodevec(x1_ref).astype(_MM)
    nv2 = nodevec(x2_ref).astype(_MM)
    adj = (lax.dot_general(nv1, nv2, _DN, preferred_element_type=jnp.float32)
           - lax.dot_general(nv2, nv1, _DN, preferred_element_type=jnp.float32))
    work_ref[...] = adj + noise_ref[0]

    def colmax(t):
        parts = []
        for c in range(16):
            w = work_ref[c * 64:(c + 1) * 64, :].reshape(8, 8, _N)
            if t is not None:
                w = jnp.where(w >= t, -jnp.inf, w)
            parts.append(jnp.max(w, axis=0))
        while len(parts) > 1:
            parts = [jnp.maximum(parts[i], parts[i + 1])
                     for i in range(0, len(parts), 2)]
        return jnp.max(parts[0], axis=0, keepdims=True).reshape(1, 1, _N)

    t20 = lax.fori_loop(1, _K, lambda _, t: colmax(t), colmax(None))
    w = work_ref[...]
    out_ref[0] = jnp.where(w >= t20.reshape(1, _N), w - noise_ref[0], 0.0)


def _run(x1, x2, noise):
    return pl.pallas_call(
        _body,
        grid=(_B,),
        in_specs=[
            pl.BlockSpec((1, _F, _N, _D), lambda b: (b, 0, 0, 0)),
            pl.BlockSpec((1, _F, _N, _D), lambda b: (b, 0, 0, 0)),
            pl.BlockSpec((1, _N, _N), lambda b: (b, 0, 0)),
        ],
        out_specs=pl.BlockSpec((1, _N, _N), lambda b: (b, 0, 0)),
        out_shape=jax.ShapeDtypeStruct((_B, _N, _N), jnp.float32),
        scratch_shapes=[pltpu.VMEM((_N, _N), jnp.float32)],
    )(x1, x2, noise)


def kernel(idx, time_in_day_feat, day_in_week_feat, emb1_table, emb2_table):
    return _run(time_in_day_feat, day_in_week_feat, _noise_const())
